# Initial kernel scaffold; baseline (speedup 1.0000x reference)
import jax
import jax.numpy as jnp
from jax.experimental import pallas as pl


def kernel(x, W):
    raise NotImplementedError("write your pallas kernel here")


# speedup vs baseline: 1.0642x; 1.0642x over previous
import jax
import jax.numpy as jnp
from jax.experimental import pallas as pl
from jax.experimental.pallas import tpu as pltpu

_D_MODEL = 4096
_N_EXPERTS = 64
_K = 8
_BLOCK = 512


def _router_body(x_ref, w_ref, logits_ref, probs_ref, idx_ref, wts_ref):
    x = x_ref[...]
    w = w_ref[...]
    logits = jax.lax.dot_general(
        x, w, (((1,), (1,)), ((), ())),
        preferred_element_type=jnp.float32,
    )
    logits_ref[...] = logits

    m = jnp.max(logits, axis=-1, keepdims=True)
    e = jnp.exp(logits - m)
    s = jnp.sum(e, axis=-1, keepdims=True)
    probs = e / s
    probs_ref[...] = probs

    cols = jax.lax.broadcasted_iota(jnp.int32, probs.shape, 1)
    p = probs
    vals = []
    idxs = []
    for _ in range(_K):
        v = jnp.max(p, axis=-1, keepdims=True)
        i = jnp.min(jnp.where(p == v, cols, _N_EXPERTS), axis=-1, keepdims=True)
        vals.append(v)
        idxs.append(i)
        p = jnp.where(cols == i, -jnp.inf, p)
    topv = jnp.concatenate(vals, axis=1)
    topi = jnp.concatenate(idxs, axis=1)
    denom = jnp.maximum(jnp.sum(topv, axis=1, keepdims=True), 1e-9)
    wts_ref[...] = topv / denom
    idx_ref[...] = topi


def kernel(x, W):
    n_tokens = x.shape[0]
    grid = (n_tokens // _BLOCK,)
    out_shape = (
        jax.ShapeDtypeStruct((n_tokens, _N_EXPERTS), jnp.float32),
        jax.ShapeDtypeStruct((n_tokens, _N_EXPERTS), jnp.float32),
        jax.ShapeDtypeStruct((n_tokens, _K), jnp.int32),
        jax.ShapeDtypeStruct((n_tokens, _K), jnp.float32),
    )
    logits, probs, topk_indices, topk_weights = pl.pallas_call(
        _router_body,
        grid=grid,
        in_specs=[
            pl.BlockSpec((_BLOCK, _D_MODEL), lambda i: (i, 0)),
            pl.BlockSpec((_N_EXPERTS, _D_MODEL), lambda i: (0, 0)),
        ],
        out_specs=(
            pl.BlockSpec((_BLOCK, _N_EXPERTS), lambda i: (i, 0)),
            pl.BlockSpec((_BLOCK, _N_EXPERTS), lambda i: (i, 0)),
            pl.BlockSpec((_BLOCK, _K), lambda i: (i, 0)),
            pl.BlockSpec((_BLOCK, _K), lambda i: (i, 0)),
        ),
        out_shape=out_shape,
        compiler_params=pltpu.CompilerParams(
            dimension_semantics=("parallel",),
        ),
    )(x, W)
    return (logits, probs, topk_indices, topk_weights)


# speedup vs baseline: 1.2144x; 1.1411x over previous
import jax
import jax.numpy as jnp
from jax.experimental import pallas as pl
from jax.experimental.pallas import tpu as pltpu

_D_MODEL = 4096
_N_EXPERTS = 64
_K = 8
_BLOCK = 1024


def _router_body(x_ref, w_ref, logits_ref, probs_ref, idx_ref, wts_ref):
    x = x_ref[...]
    w = w_ref[...]
    logits = jax.lax.dot_general(
        x, w, (((1,), (1,)), ((), ())),
        preferred_element_type=jnp.float32,
    )
    logits_ref[...] = logits

    m = jnp.max(logits, axis=-1, keepdims=True)
    e = jnp.exp(logits - m)
    s = jnp.sum(e, axis=-1, keepdims=True)
    probs = e / s
    probs_ref[...] = probs

    cols = jax.lax.broadcasted_iota(jnp.int32, probs.shape, 1)
    p = probs
    vals = []
    idxs = []
    for _ in range(_K):
        v = jnp.max(p, axis=-1, keepdims=True)
        i = jnp.min(jnp.where(p == v, cols, _N_EXPERTS), axis=-1, keepdims=True)
        vals.append(v)
        idxs.append(i)
        p = jnp.where(cols == i, -jnp.inf, p)
    topv = jnp.concatenate(vals, axis=1)
    topi = jnp.concatenate(idxs, axis=1)
    denom = jnp.maximum(jnp.sum(topv, axis=1, keepdims=True), 1e-9)
    wts_ref[...] = topv / denom
    idx_ref[...] = topi


def kernel(x, W):
    n_tokens = x.shape[0]
    grid = (n_tokens // _BLOCK,)
    out_shape = (
        jax.ShapeDtypeStruct((n_tokens, _N_EXPERTS), jnp.float32),
        jax.ShapeDtypeStruct((n_tokens, _N_EXPERTS), jnp.float32),
        jax.ShapeDtypeStruct((n_tokens, _K), jnp.int32),
        jax.ShapeDtypeStruct((n_tokens, _K), jnp.float32),
    )
    logits, probs, topk_indices, topk_weights = pl.pallas_call(
        _router_body,
        grid=grid,
        in_specs=[
            pl.BlockSpec((_BLOCK, _D_MODEL), lambda i: (i, 0)),
            pl.BlockSpec((_N_EXPERTS, _D_MODEL), lambda i: (0, 0)),
        ],
        out_specs=(
            pl.BlockSpec((_BLOCK, _N_EXPERTS), lambda i: (i, 0)),
            pl.BlockSpec((_BLOCK, _N_EXPERTS), lambda i: (i, 0)),
            pl.BlockSpec((_BLOCK, _K), lambda i: (i, 0)),
            pl.BlockSpec((_BLOCK, _K), lambda i: (i, 0)),
        ),
        out_shape=out_shape,
        compiler_params=pltpu.CompilerParams(
            dimension_semantics=("parallel",),
        ),
    )(x, W)
    return (logits, probs, topk_indices, topk_weights)


# speedup vs baseline: 1.3599x; 1.1198x over previous
import jax
import jax.numpy as jnp
from jax.experimental import pallas as pl
from jax.experimental.pallas import tpu as pltpu

_D_MODEL = 4096
_N_EXPERTS = 64
_K = 8
_BLOCK = 1024


def _router_body(x_ref, w_ref, logits_ref, probs_ref, idx_ref, wts_ref):
    x = x_ref[...]
    w = w_ref[...]
    logits = jax.lax.dot_general(
        x, w, (((1,), (1,)), ((), ())),
        preferred_element_type=jnp.float32,
    )
    logits_ref[...] = logits

    m = jnp.max(logits, axis=-1, keepdims=True)
    e = jnp.exp(logits - m)
    s = jnp.sum(e, axis=-1, keepdims=True)
    probs = e / s
    probs_ref[...] = probs

    cols = jax.lax.broadcasted_iota(jnp.int32, probs.shape, 1)
    bits = jax.lax.bitcast_convert_type(probs, jnp.int32)
    key = jax.lax.bitcast_convert_type(
        (bits & ~jnp.int32(_N_EXPERTS - 1)) | (_N_EXPERTS - 1 - cols),
        jnp.float32,
    )
    maxes = []
    for _ in range(_K):
        m = jnp.max(key, axis=-1, keepdims=True)
        key = jnp.where(key == m, -jnp.inf, key)
        maxes.append(m)
    mk = jax.lax.bitcast_convert_type(
        jnp.concatenate(maxes, axis=1), jnp.int32)
    topi = (_N_EXPERTS - 1) - (mk & jnp.int32(_N_EXPERTS - 1))
    topv = jax.lax.bitcast_convert_type(
        mk & ~jnp.int32(_N_EXPERTS - 1), jnp.float32)
    denom = jnp.maximum(jnp.sum(topv, axis=1, keepdims=True), 1e-9)
    wts_ref[...] = topv / denom
    idx_ref[...] = topi


def kernel(x, W):
    n_tokens = x.shape[0]
    grid = (n_tokens // _BLOCK,)
    out_shape = (
        jax.ShapeDtypeStruct((n_tokens, _N_EXPERTS), jnp.float32),
        jax.ShapeDtypeStruct((n_tokens, _N_EXPERTS), jnp.float32),
        jax.ShapeDtypeStruct((n_tokens, _K), jnp.int32),
        jax.ShapeDtypeStruct((n_tokens, _K), jnp.float32),
    )
    logits, probs, topk_indices, topk_weights = pl.pallas_call(
        _router_body,
        grid=grid,
        in_specs=[
            pl.BlockSpec((_BLOCK, _D_MODEL), lambda i: (i, 0)),
            pl.BlockSpec((_N_EXPERTS, _D_MODEL), lambda i: (0, 0)),
        ],
        out_specs=(
            pl.BlockSpec((_BLOCK, _N_EXPERTS), lambda i: (i, 0)),
            pl.BlockSpec((_BLOCK, _N_EXPERTS), lambda i: (i, 0)),
            pl.BlockSpec((_BLOCK, _K), lambda i: (i, 0)),
            pl.BlockSpec((_BLOCK, _K), lambda i: (i, 0)),
        ),
        out_shape=out_shape,
        compiler_params=pltpu.CompilerParams(
            dimension_semantics=("parallel",),
        ),
    )(x, W)
    return (logits, probs, topk_indices, topk_weights)
